# baseline (speedup 1.0000x reference)
import jax
import jax.numpy as jnp
from jax.experimental import pallas as pl
from jax.experimental.pallas import tpu as pltpu

_BEV_EXTENT = (-32.0, -32.0, 32.0, 32.0)
_EPS = 1e-8
_Q = 1024


def _body(qsrc_ref, fsrc_ref, tgt_ref, tn_ref, ftgt_ref, sflow_ref, misc_ref,
          trafo_ref, out_ref):
    g = pl.program_id(0)
    q = pl.program_id(1)

    p = qsrc_ref[0]
    fsrc = fsrc_ref[0]
    w = p + fsrc

    gram = jax.lax.dot_general(w, tgt_ref[0], (((1,), (1,)), ((), ())),
                               preferred_element_type=jnp.float32)
    u = tn_ref[0] - 2.0 * gram
    umin = jnp.min(u, axis=1, keepdims=True)
    wn = jnp.sum(w * w, axis=1, keepdims=True)
    nn_d2 = jnp.maximum(wn + umin, 0.0)
    onehot = (u == umin).astype(jnp.float32)
    fnn = jax.lax.dot_general(onehot, ftgt_ref[0], (((1,), (0,)), ((), ())),
                              preferred_element_type=jnp.float32)

    x0, y0, x1, y1 = _BEV_EXTENT
    wx = w[:, 0:1]
    wy = w[:, 1:2]
    in_bev = ((wx >= x0) & (wx <= x1) & (wy >= y0) & (wy <= y1)
              ).astype(jnp.float32)

    knn_num = jnp.sum(in_bev * nn_d2)
    knn_den = jnp.sum(in_bev)
    opp = fsrc + fnn
    opp_err = jnp.sum(opp * opp, axis=1, keepdims=True)
    opp_num = jnp.sum(in_bev * opp_err)

    trafo = trafo_ref[g]
    rot = trafo[:3, :3]
    trans = trafo[:3, 3]
    rp = jax.lax.dot_general(p, rot, (((1,), (1,)), ((), ())),
                             preferred_element_type=jnp.float32)
    trafo_flow = rp + trans[None, :] - p
    serr = sflow_ref[0] - trafo_flow
    serr2 = jnp.sum(serr * serr, axis=1, keepdims=True)
    staticness = misc_ref[0][:, 0:1]
    static_num = jnp.sum(staticness * serr2)
    static_den = jnp.sum(staticness)

    dis = misc_ref[0][:, 1:2]
    valid = (dis == dis).astype(jnp.float32)
    occ_sum = jnp.sum(jnp.where(dis == dis, dis, 0.0))
    occ_cnt = jnp.sum(valid)

    nb = trafo_ref.shape[0] // 2
    eye = jnp.eye(4, dtype=jnp.float32)
    sse = jnp.float32(0.0)
    for b in range(nb):
        comp = jnp.dot(trafo_ref[b], trafo_ref[nb + b],
                       preferred_element_type=jnp.float32)
        dlt = comp - eye
        sse = sse + jnp.sum(dlt * dlt)
    first = jnp.logical_and(g == 0, q == 0).astype(jnp.float32)
    sse = sse * first

    slots = jax.lax.broadcasted_iota(jnp.int32, (1, 1, 8), 2)
    vals = [knn_num, knn_den, opp_num, static_num, static_den, occ_sum,
            occ_cnt, sse]
    row = jnp.zeros((1, 1, 8), jnp.float32)
    for k, v in enumerate(vals):
        row = row + jnp.where(slots == k, v, 0.0)

    @pl.when(q == 0)
    def _():
        out_ref[...] = row

    @pl.when(q != 0)
    def _():
        out_ref[...] += row


def kernel(pc0, pc1, fw_aggregated_flow, bw_aggregated_flow, fw_static_flow,
           bw_static_flow, fw_static_aggr_trafo, bw_static_aggr_trafo,
           fw_staticness, bw_staticness, fw_disappearing, bw_disappearing):
    B, N, _ = pc0.shape
    G = 2 * B

    qsrc = jnp.concatenate([pc0, pc1], axis=0)
    tgt = jnp.concatenate([pc1, pc0], axis=0)
    tn = jnp.concatenate([jnp.sum(pc1 * pc1, axis=-1),
                          jnp.sum(pc0 * pc0, axis=-1)],
                         axis=0)[:, None, :]
    fsrc = jnp.concatenate([fw_aggregated_flow, bw_aggregated_flow], axis=0)
    ftgt = jnp.concatenate([bw_aggregated_flow, fw_aggregated_flow], axis=0)
    sflow = jnp.concatenate([fw_static_flow, bw_static_flow], axis=0)
    misc = jnp.stack([jnp.concatenate([fw_staticness, bw_staticness], axis=0),
                      jnp.concatenate([fw_disappearing, bw_disappearing],
                                      axis=0)], axis=-1)
    trafos = jnp.concatenate([fw_static_aggr_trafo, bw_static_aggr_trafo],
                             axis=0)

    nq = N // _Q
    out = pl.pallas_call(
        _body,
        grid=(G, nq),
        in_specs=[
            pl.BlockSpec((1, _Q, 3), lambda g, q: (g, q, 0)),
            pl.BlockSpec((1, _Q, 3), lambda g, q: (g, q, 0)),
            pl.BlockSpec((1, N, 3), lambda g, q: (g, 0, 0)),
            pl.BlockSpec((1, 1, N), lambda g, q: (g, 0, 0)),
            pl.BlockSpec((1, N, 3), lambda g, q: (g, 0, 0)),
            pl.BlockSpec((1, _Q, 3), lambda g, q: (g, q, 0)),
            pl.BlockSpec((1, _Q, 2), lambda g, q: (g, q, 0)),
            pl.BlockSpec((G, 4, 4), lambda g, q: (0, 0, 0)),
        ],
        out_specs=pl.BlockSpec((1, 1, 8), lambda g, q: (g, 0, 0)),
        out_shape=jax.ShapeDtypeStruct((G, 1, 8), jnp.float32),
        compiler_params=pltpu.CompilerParams(
            dimension_semantics=("parallel", "arbitrary")),
    )(qsrc, fsrc, tgt, tn, ftgt, sflow, misc, trafos)

    out = out.reshape(G, 8)
    fw = out[:B]
    bw = out[B:]
    eps = jnp.float32(_EPS)

    def seg(rows):
        s = jnp.sum(rows, axis=0)
        den = s[1] + eps
        return s[0] / den, s[2] / den, s[3] / (s[4] + eps)

    fw_fl, fw_opp, fw_static = seg(fw)
    bw_fl, bw_opp, bw_static = seg(bw)
    flow_loss = 0.5 * (fw_fl + bw_fl)
    opposite_flow_loss = 0.5 * (fw_opp + bw_opp)
    static_flow_loss = 0.5 * (fw_static + bw_static)
    occlusion_loss = jnp.sum(out[:, 5]) / (jnp.sum(out[:, 6]) + eps)
    trafo_loss = jnp.sum(out[:, 7]) / (B * 16.0)

    total = (static_flow_loss + trafo_loss + 0.1 * occlusion_loss
             + flow_loss + opposite_flow_loss)
    return total


# speedup vs baseline: 1.0335x; 1.0335x over previous
import jax
import jax.numpy as jnp
from jax.experimental import pallas as pl
from jax.experimental.pallas import tpu as pltpu

_BEV_EXTENT = (-32.0, -32.0, 32.0, 32.0)
_EPS = 1e-8
_Q = 1024


def _body(qsrc_ref, fsrc_ref, tgt_ref, tn_ref, ftgt_ref, sflow_ref, misc_ref,
          trafo_ref, out_ref):
    g = pl.program_id(0)
    q = pl.program_id(1)

    p = qsrc_ref[0]
    fsrc = fsrc_ref[0]
    w = p + fsrc

    gram = jax.lax.dot_general(w, tgt_ref[0], (((1,), (0,)), ((), ())),
                               preferred_element_type=jnp.float32)
    u = tn_ref[0] - 2.0 * gram
    umin = jnp.min(u, axis=1, keepdims=True)
    wn = jnp.sum(w * w, axis=1, keepdims=True)
    nn_d2 = jnp.maximum(wn + umin, 0.0)
    onehot = (u == umin).astype(jnp.float32)
    fnn = jax.lax.dot_general(onehot, ftgt_ref[0], (((1,), (0,)), ((), ())),
                              preferred_element_type=jnp.float32)

    x0, y0, x1, y1 = _BEV_EXTENT
    wx = w[:, 0:1]
    wy = w[:, 1:2]
    in_bev = ((wx >= x0) & (wx <= x1) & (wy >= y0) & (wy <= y1)
              ).astype(jnp.float32)

    knn_num = jnp.sum(in_bev * nn_d2)
    knn_den = jnp.sum(in_bev)
    opp = fsrc + fnn
    opp_err = jnp.sum(opp * opp, axis=1, keepdims=True)
    opp_num = jnp.sum(in_bev * opp_err)

    trafo = trafo_ref[g]
    rot = trafo[:3, :3]
    trans = trafo[:3, 3]
    rp = jax.lax.dot_general(p, rot, (((1,), (1,)), ((), ())),
                             preferred_element_type=jnp.float32)
    trafo_flow = rp + trans[None, :] - p
    serr = sflow_ref[0] - trafo_flow
    serr2 = jnp.sum(serr * serr, axis=1, keepdims=True)
    staticness = misc_ref[0][:, 0:1]
    static_num = jnp.sum(staticness * serr2)
    static_den = jnp.sum(staticness)

    dis = misc_ref[0][:, 1:2]
    valid = (dis == dis).astype(jnp.float32)
    occ_sum = jnp.sum(jnp.where(dis == dis, dis, 0.0))
    occ_cnt = jnp.sum(valid)

    nb = trafo_ref.shape[0] // 2
    eye = jnp.eye(4, dtype=jnp.float32)
    sse = jnp.float32(0.0)
    for b in range(nb):
        comp = jnp.dot(trafo_ref[b], trafo_ref[nb + b],
                       preferred_element_type=jnp.float32)
        dlt = comp - eye
        sse = sse + jnp.sum(dlt * dlt)
    first = jnp.logical_and(g == 0, q == 0).astype(jnp.float32)
    sse = sse * first

    slots = jax.lax.broadcasted_iota(jnp.int32, (1, 1, 8), 2)
    vals = [knn_num, knn_den, opp_num, static_num, static_den, occ_sum,
            occ_cnt, sse]
    row = jnp.zeros((1, 1, 8), jnp.float32)
    for k, v in enumerate(vals):
        row = row + jnp.where(slots == k, v, 0.0)

    @pl.when(q == 0)
    def _():
        out_ref[...] = row

    @pl.when(q != 0)
    def _():
        out_ref[...] += row


def kernel(pc0, pc1, fw_aggregated_flow, bw_aggregated_flow, fw_static_flow,
           bw_static_flow, fw_static_aggr_trafo, bw_static_aggr_trafo,
           fw_staticness, bw_staticness, fw_disappearing, bw_disappearing):
    B, N, _ = pc0.shape
    G = 2 * B

    qsrc = jnp.concatenate([pc0, pc1], axis=0)
    tgt = jnp.concatenate([pc1, pc0], axis=0).transpose(0, 2, 1)
    tn = jnp.concatenate([jnp.sum(pc1 * pc1, axis=-1),
                          jnp.sum(pc0 * pc0, axis=-1)],
                         axis=0)[:, None, :]
    fsrc = jnp.concatenate([fw_aggregated_flow, bw_aggregated_flow], axis=0)
    ftgt = jnp.concatenate([bw_aggregated_flow, fw_aggregated_flow], axis=0)
    sflow = jnp.concatenate([fw_static_flow, bw_static_flow], axis=0)
    misc = jnp.stack([jnp.concatenate([fw_staticness, bw_staticness], axis=0),
                      jnp.concatenate([fw_disappearing, bw_disappearing],
                                      axis=0)], axis=-1)
    trafos = jnp.concatenate([fw_static_aggr_trafo, bw_static_aggr_trafo],
                             axis=0)

    nq = N // _Q
    out = pl.pallas_call(
        _body,
        grid=(G, nq),
        in_specs=[
            pl.BlockSpec((1, _Q, 3), lambda g, q: (g, q, 0)),
            pl.BlockSpec((1, _Q, 3), lambda g, q: (g, q, 0)),
            pl.BlockSpec((1, 3, N), lambda g, q: (g, 0, 0)),
            pl.BlockSpec((1, 1, N), lambda g, q: (g, 0, 0)),
            pl.BlockSpec((1, N, 3), lambda g, q: (g, 0, 0)),
            pl.BlockSpec((1, _Q, 3), lambda g, q: (g, q, 0)),
            pl.BlockSpec((1, _Q, 2), lambda g, q: (g, q, 0)),
            pl.BlockSpec((G, 4, 4), lambda g, q: (0, 0, 0)),
        ],
        out_specs=pl.BlockSpec((1, 1, 8), lambda g, q: (g, 0, 0)),
        out_shape=jax.ShapeDtypeStruct((G, 1, 8), jnp.float32),
        compiler_params=pltpu.CompilerParams(
            dimension_semantics=("parallel", "arbitrary")),
    )(qsrc, fsrc, tgt, tn, ftgt, sflow, misc, trafos)

    out = out.reshape(G, 8)
    fw = out[:B]
    bw = out[B:]
    eps = jnp.float32(_EPS)

    def seg(rows):
        s = jnp.sum(rows, axis=0)
        den = s[1] + eps
        return s[0] / den, s[2] / den, s[3] / (s[4] + eps)

    fw_fl, fw_opp, fw_static = seg(fw)
    bw_fl, bw_opp, bw_static = seg(bw)
    flow_loss = 0.5 * (fw_fl + bw_fl)
    opposite_flow_loss = 0.5 * (fw_opp + bw_opp)
    static_flow_loss = 0.5 * (fw_static + bw_static)
    occlusion_loss = jnp.sum(out[:, 5]) / (jnp.sum(out[:, 6]) + eps)
    trafo_loss = jnp.sum(out[:, 7]) / (B * 16.0)

    total = (static_flow_loss + trafo_loss + 0.1 * occlusion_loss
             + flow_loss + opposite_flow_loss)
    return total


# speedup vs baseline: 1.0731x; 1.0383x over previous
import jax
import jax.numpy as jnp
from jax.experimental import pallas as pl
from jax.experimental.pallas import tpu as pltpu

_BEV_EXTENT = (-32.0, -32.0, 32.0, 32.0)
_EPS = 1e-8
_Q = 512


def _body(a0_ref, a1_ref, pc0T_ref, pc1T_ref, tn0_ref, tn1_ref,
          fwt_ref, bwt_ref, out_ref):
    g = pl.program_id(0)
    nb = fwt_ref.shape[0]
    is_fw = g < nb

    src = jnp.where(is_fw, a0_ref[0], a1_ref[0])
    tgt = jnp.where(is_fw, a1_ref[0], a0_ref[0])
    tT = jnp.where(is_fw, pc1T_ref[0], pc0T_ref[0])
    tn = jnp.where(is_fw, tn1_ref[0], tn0_ref[0])
    ftgt = tgt[:, 3:6]

    b = jax.lax.rem(g, nb)
    trafo = jnp.where(is_fw, fwt_ref[b], bwt_ref[b])
    rot = trafo[:3, :3]
    trans = trafo[:3, 3]

    x0, y0, x1, y1 = _BEV_EXTENT
    n = src.shape[0]
    acc = [jnp.float32(0.0)] * 7

    for c in range(n // _Q):
        sl = slice(c * _Q, (c + 1) * _Q)
        blk = src[sl, :]
        p = blk[:, 0:3]
        fsrc = blk[:, 3:6]
        w = p + fsrc

        gram = jax.lax.dot_general(w, tT, (((1,), (0,)), ((), ())),
                                   preferred_element_type=jnp.float32)
        u = tn - 2.0 * gram
        umin = jnp.min(u, axis=1, keepdims=True)
        wn = jnp.sum(w * w, axis=1, keepdims=True)
        nn_d2 = jnp.maximum(wn + umin, 0.0)
        onehot = (u == umin).astype(jnp.float32)
        fnn = jax.lax.dot_general(onehot, ftgt, (((1,), (0,)), ((), ())),
                                  preferred_element_type=jnp.float32)

        wx = w[:, 0:1]
        wy = w[:, 1:2]
        in_bev = ((wx >= x0) & (wx <= x1) & (wy >= y0) & (wy <= y1)
                  ).astype(jnp.float32)

        opp = fsrc + fnn
        opp_err = jnp.sum(opp * opp, axis=1, keepdims=True)

        rp = jax.lax.dot_general(p, rot, (((1,), (1,)), ((), ())),
                                 preferred_element_type=jnp.float32)
        trafo_flow = rp + trans[None, :] - p
        serr = blk[:, 6:9] - trafo_flow
        serr2 = jnp.sum(serr * serr, axis=1, keepdims=True)
        sn = blk[:, 9:10]

        dis = blk[:, 10:11]
        valid = (dis == dis).astype(jnp.float32)

        acc[0] += jnp.sum(in_bev * nn_d2)
        acc[1] += jnp.sum(in_bev)
        acc[2] += jnp.sum(in_bev * opp_err)
        acc[3] += jnp.sum(sn * serr2)
        acc[4] += jnp.sum(sn)
        acc[5] += jnp.sum(jnp.where(dis == dis, dis, 0.0))
        acc[6] += jnp.sum(valid)

    eye = jnp.eye(4, dtype=jnp.float32)
    sse = jnp.float32(0.0)
    for bb in range(nb):
        comp = jnp.dot(fwt_ref[bb], bwt_ref[bb],
                       preferred_element_type=jnp.float32)
        dlt = comp - eye
        sse = sse + jnp.sum(dlt * dlt)
    sse = sse * (g == 0).astype(jnp.float32)

    slots = jax.lax.broadcasted_iota(jnp.int32, (1, 1, 8), 2)
    vals = acc + [sse]
    row = jnp.zeros((1, 1, 8), jnp.float32)
    for k, v in enumerate(vals):
        row = row + jnp.where(slots == k, v, 0.0)
    out_ref[...] = row


def kernel(pc0, pc1, fw_aggregated_flow, bw_aggregated_flow, fw_static_flow,
           bw_static_flow, fw_static_aggr_trafo, bw_static_aggr_trafo,
           fw_staticness, bw_staticness, fw_disappearing, bw_disappearing):
    B, N, _ = pc0.shape
    G = 2 * B

    a0 = jnp.concatenate(
        [pc0, fw_aggregated_flow, fw_static_flow,
         fw_staticness[..., None], fw_disappearing[..., None]], axis=-1)
    a1 = jnp.concatenate(
        [pc1, bw_aggregated_flow, bw_static_flow,
         bw_staticness[..., None], bw_disappearing[..., None]], axis=-1)

    pc0T = pc0.transpose(0, 2, 1)
    pc1T = pc1.transpose(0, 2, 1)
    tn0 = jnp.sum(pc0 * pc0, axis=-1)[:, None, :]
    tn1 = jnp.sum(pc1 * pc1, axis=-1)[:, None, :]

    bspec = lambda shape: pl.BlockSpec(shape, lambda g: (jax.lax.rem(g, B),)
                                       + (0,) * (len(shape) - 1))

    out = pl.pallas_call(
        _body,
        grid=(G,),
        in_specs=[
            bspec((1, N, 11)),
            bspec((1, N, 11)),
            bspec((1, 3, N)),
            bspec((1, 3, N)),
            bspec((1, 1, N)),
            bspec((1, 1, N)),
            pl.BlockSpec((B, 4, 4), lambda g: (0, 0, 0)),
            pl.BlockSpec((B, 4, 4), lambda g: (0, 0, 0)),
        ],
        out_specs=pl.BlockSpec((1, 1, 8), lambda g: (g, 0, 0)),
        out_shape=jax.ShapeDtypeStruct((G, 1, 8), jnp.float32),
        compiler_params=pltpu.CompilerParams(
            dimension_semantics=("arbitrary",)),
    )(a0, a1, pc0T, pc1T, tn0, tn1,
      fw_static_aggr_trafo, bw_static_aggr_trafo)

    out = out.reshape(G, 8)
    fw = out[:B]
    bw = out[B:]
    eps = jnp.float32(_EPS)

    def seg(rows):
        s = jnp.sum(rows, axis=0)
        den = s[1] + eps
        return s[0] / den, s[2] / den, s[3] / (s[4] + eps)

    fw_fl, fw_opp, fw_static = seg(fw)
    bw_fl, bw_opp, bw_static = seg(bw)
    flow_loss = 0.5 * (fw_fl + bw_fl)
    opposite_flow_loss = 0.5 * (fw_opp + bw_opp)
    static_flow_loss = 0.5 * (fw_static + bw_static)
    occlusion_loss = jnp.sum(out[:, 5]) / (jnp.sum(out[:, 6]) + eps)
    trafo_loss = jnp.sum(out[:, 7]) / (B * 16.0)

    total = (static_flow_loss + trafo_loss + 0.1 * occlusion_loss
             + flow_loss + opposite_flow_loss)
    return total
